# Initial kernel scaffold; baseline (speedup 1.0000x reference)
import jax
import jax.numpy as jnp
from jax.experimental import pallas as pl


def kernel(x, edge_index, W1a, b1a, W1b, b1b, g1, be1, W2a, b2a, W2b, b2b, g2, be2, W5a, b5a, W5b, b5b, g5, be5):
    raise NotImplementedError("write your pallas kernel here")


# speedup vs baseline: 2.9668x; 2.9668x over previous
import functools

import jax
import jax.numpy as jnp
from jax import lax
from jax.experimental import pallas as pl
from jax.experimental.pallas import tpu as pltpu
from jax.experimental.pallas import tpu_sc as plsc

N = 10000
D = 128
E = 320000
BN_EPS = 1e-5

NC = 2
NS = 16
NW = NC * NS
CHUNK = 128
GPT = 80
EPAD = NW * GPT * CHUNK
NACC = 10240
ZROWS = NACC // NS
OROWS = NACC // NS

def _sc_agg_body(h_hbm, src_hbm, dst_hbm, zeros_hbm, out_hbm,
                 src_v, dst_v, rows_v, acc, sem):
    c = lax.axis_index("c")
    s = lax.axis_index("s")
    w = s * NC + c

    pltpu.sync_copy(zeros_hbm.at[pl.ds(s * ZROWS, ZROWS)],
                    acc.at[pl.ds(s * ZROWS, ZROWS)])
    pltpu.sync_copy(src_hbm.at[pl.ds(w * GPT, GPT)], src_v)
    pltpu.sync_copy(dst_hbm.at[pl.ds(w * GPT, GPT)], dst_v)
    plsc.subcore_barrier()

    def body(g, carry):
        pltpu.async_copy(h_hbm.at[src_v.at[g]], rows_v, sem).wait()
        pltpu.sync_copy(rows_v, acc.at[dst_v.at[g]], add=True)
        return carry

    lax.fori_loop(0, GPT, body, 0)
    plsc.subcore_barrier()

    pltpu.sync_copy(acc.at[pl.ds(s * OROWS, OROWS)],
                    out_hbm.at[c, pl.ds(s * OROWS, OROWS)])


@functools.cache
def _sc_agg_call():
    mesh = plsc.VectorSubcoreMesh(core_axis_name="c", subcore_axis_name="s",
                                  num_cores=NC, num_subcores=NS)
    return pl.kernel(
        _sc_agg_body,
        out_type=jax.ShapeDtypeStruct((NC, NACC, D), jnp.float32),
        mesh=mesh,
        scratch_types=[
            pltpu.VMEM((GPT, CHUNK), jnp.int32),
            pltpu.VMEM((GPT, CHUNK), jnp.int32),
            pltpu.VMEM((CHUNK, D), jnp.float32),
            pltpu.VMEM_SHARED((NACC, D), jnp.float32),
            pltpu.SemaphoreType.DMA,
        ],
    )


def _dense_body(x_ref, a_ref, wa_ref, ba_ref, wb_ref, bb_ref, g_ref, be_ref,
                out_ref):
    z = x_ref[...] + a_ref[0, :N] + a_ref[1, :N]
    t = jnp.maximum(
        jnp.dot(z, wa_ref[...], preferred_element_type=jnp.float32)
        + ba_ref[...], 0.0)
    u = (jnp.dot(t, wb_ref[...], preferred_element_type=jnp.float32)
         + bb_ref[...])
    v = jnp.maximum(u, 0.0)
    mu = jnp.mean(v, axis=0, keepdims=True)
    var = jnp.mean((v - mu) ** 2, axis=0, keepdims=True)
    out_ref[...] = (g_ref[...] * (v - mu) * lax.rsqrt(var + BN_EPS)
                    + be_ref[...])


_dense_call = pl.pallas_call(
    _dense_body,
    out_shape=jax.ShapeDtypeStruct((N, D), jnp.float32),
)


def kernel(x, edge_index, W1a, b1a, W1b, b1b, g1, be1,
           W2a, b2a, W2b, b2b, g2, be2,
           W5a, b5a, W5b, b5b, g5, be5):
    src = edge_index[0]
    dst = edge_index[1]
    pad = EPAD - E
    src_p = jnp.concatenate(
        [src, jnp.zeros((pad,), jnp.int32)]).reshape(EPAD // CHUNK, CHUNK)
    dst_p = jnp.concatenate(
        [dst, jnp.full((pad,), N, jnp.int32)]).reshape(EPAD // CHUNK, CHUNK)
    zeros = jnp.zeros((NACC, D), jnp.float32)

    W5b_p = jnp.pad(W5b, ((0, 0), (0, D - W5b.shape[1])))
    b5b_p = jnp.pad(b5b, (0, D - b5b.shape[0]))
    g5_p = jnp.pad(g5, (0, D - g5.shape[0]))
    be5_p = jnp.pad(be5, (0, D - be5.shape[0]))

    h = x
    layers = [
        (W1a, b1a, W1b, b1b, g1, be1),
        (W2a, b2a, W2b, b2b, g2, be2),
        (W5a, b5a, W5b_p, b5b_p, g5_p, be5_p),
    ]
    for Wa, ba, Wb, bb, g, be in layers:
        agg = _sc_agg_call()(h, src_p, dst_p, zeros)
        h = _dense_call(h, agg, Wa, ba.reshape(1, D), Wb, bb.reshape(1, D),
                        g.reshape(1, D), be.reshape(1, D))
    return h[:, :2]


# speedup vs baseline: 3.2024x; 1.0794x over previous
import functools

import jax
import jax.numpy as jnp
from jax import lax
from jax.experimental import pallas as pl
from jax.experimental.pallas import tpu as pltpu
from jax.experimental.pallas import tpu_sc as plsc

N = 10000
D = 128
E = 320000
BN_EPS = 1e-5

NC = 2
NS = 16
NW = NC * NS
CHUNK = 128
GPT = 80
PHASE = 40
EPAD = NW * GPT * CHUNK
NACC = 10240
ZROWS = NACC // NS
OROWS = NACC // NS

def _sc_agg_body(h_hbm, src_hbm, dst_hbm, zeros_hbm, out_hbm,
                 src_v, dst_v, rows0, rows1, acc,
                 gsem0, gsem1, ssem0, ssem1):
    c = lax.axis_index("c")
    s = lax.axis_index("s")
    w = s * NC + c

    pltpu.sync_copy(zeros_hbm.at[pl.ds(s * ZROWS, ZROWS)],
                    acc.at[pl.ds(s * ZROWS, ZROWS)])
    plsc.subcore_barrier()

    def g_issue(buf, sem, chunk):
        pltpu.async_copy(h_hbm.at[src_v.at[chunk]], buf, sem)

    def g_wait(buf, sem):
        pltpu.make_async_copy(h_hbm.at[src_v.at[0]], buf, sem).wait()

    def s_issue(buf, sem, chunk):
        pltpu.async_copy(buf, acc.at[dst_v.at[chunk]], sem, add=True)

    def s_wait(buf, sem):
        pltpu.make_async_copy(buf, acc.at[dst_v.at[0]], sem).wait()

    def phase(off):
        pltpu.sync_copy(src_hbm.at[pl.ds(w * GPT + off, PHASE)], src_v)
        pltpu.sync_copy(dst_hbm.at[pl.ds(w * GPT + off, PHASE)], dst_v)

        g_issue(rows0, gsem0, 0)
        g_wait(rows0, gsem0)
        g_issue(rows1, gsem1, 1)
        s_issue(rows0, ssem0, 0)

        def body(j, carry):
            g_wait(rows1, gsem1)
            s_issue(rows1, ssem1, 2 * j + 1)
            s_wait(rows0, ssem0)
            g_issue(rows0, gsem0, 2 * j + 2)
            g_wait(rows0, gsem0)
            s_issue(rows0, ssem0, 2 * j + 2)
            s_wait(rows1, ssem1)
            g_issue(rows1, gsem1, 2 * j + 3)
            return carry

        lax.fori_loop(0, PHASE // 2 - 1, body, 0)
        g_wait(rows1, gsem1)
        s_issue(rows1, ssem1, PHASE - 1)
        s_wait(rows0, ssem0)
        s_wait(rows1, ssem1)

    phase(0)
    phase(PHASE)
    plsc.subcore_barrier()

    pltpu.sync_copy(acc.at[pl.ds(s * OROWS, OROWS)],
                    out_hbm.at[c, pl.ds(s * OROWS, OROWS)])


@functools.cache
def _sc_agg_call():
    mesh = plsc.VectorSubcoreMesh(core_axis_name="c", subcore_axis_name="s",
                                  num_cores=NC, num_subcores=NS)
    return pl.kernel(
        _sc_agg_body,
        out_type=jax.ShapeDtypeStruct((NC, NACC, D), jnp.float32),
        mesh=mesh,
        scratch_types=[
            pltpu.VMEM((PHASE, CHUNK), jnp.int32),
            pltpu.VMEM((PHASE, CHUNK), jnp.int32),
            pltpu.VMEM((CHUNK, D), jnp.float32),
            pltpu.VMEM((CHUNK, D), jnp.float32),
            pltpu.VMEM_SHARED((NACC, D), jnp.float32),
            pltpu.SemaphoreType.DMA,
            pltpu.SemaphoreType.DMA,
            pltpu.SemaphoreType.DMA,
            pltpu.SemaphoreType.DMA,
        ],
    )


def _dense_body(x_ref, a_ref, wa_ref, ba_ref, wb_ref, bb_ref, g_ref, be_ref,
                out_ref):
    z = x_ref[...] + a_ref[0, :N] + a_ref[1, :N]
    t = jnp.maximum(
        jnp.dot(z, wa_ref[...], preferred_element_type=jnp.float32)
        + ba_ref[...], 0.0)
    u = (jnp.dot(t, wb_ref[...], preferred_element_type=jnp.float32)
         + bb_ref[...])
    v = jnp.maximum(u, 0.0)
    mu = jnp.mean(v, axis=0, keepdims=True)
    var = jnp.mean((v - mu) ** 2, axis=0, keepdims=True)
    out_ref[...] = (g_ref[...] * (v - mu) * lax.rsqrt(var + BN_EPS)
                    + be_ref[...])


_dense_call = pl.pallas_call(
    _dense_body,
    out_shape=jax.ShapeDtypeStruct((N, D), jnp.float32),
)


def kernel(x, edge_index, W1a, b1a, W1b, b1b, g1, be1,
           W2a, b2a, W2b, b2b, g2, be2,
           W5a, b5a, W5b, b5b, g5, be5):
    src = edge_index[0]
    dst = edge_index[1]
    pad = EPAD - E
    src_p = jnp.concatenate(
        [src, jnp.zeros((pad,), jnp.int32)]).reshape(EPAD // CHUNK, CHUNK)
    dst_p = jnp.concatenate(
        [dst, jnp.full((pad,), N, jnp.int32)]).reshape(EPAD // CHUNK, CHUNK)
    zeros = jnp.zeros((NACC, D), jnp.float32)

    W5b_p = jnp.pad(W5b, ((0, 0), (0, D - W5b.shape[1])))
    b5b_p = jnp.pad(b5b, (0, D - b5b.shape[0]))
    g5_p = jnp.pad(g5, (0, D - g5.shape[0]))
    be5_p = jnp.pad(be5, (0, D - be5.shape[0]))

    h = x
    layers = [
        (W1a, b1a, W1b, b1b, g1, be1),
        (W2a, b2a, W2b, b2b, g2, be2),
        (W5a, b5a, W5b_p, b5b_p, g5_p, be5_p),
    ]
    for Wa, ba, Wb, bb, g, be in layers:
        agg = _sc_agg_call()(h, src_p, dst_p, zeros)
        h = _dense_call(h, agg, Wa, ba.reshape(1, D), Wb, bb.reshape(1, D),
                        g.reshape(1, D), be.reshape(1, D))
    return h[:, :2]
